# Initial kernel scaffold; baseline (speedup 1.0000x reference)
import jax
import jax.numpy as jnp
from jax.experimental import pallas as pl


def kernel(indices, table):
    raise NotImplementedError("write your pallas kernel here")


# speedup vs baseline: 1.5619x; 1.5619x over previous
import functools

import jax
import jax.numpy as jnp
from jax import lax
from jax.experimental import pallas as pl
from jax.experimental.pallas import tpu as pltpu
from jax.experimental.pallas import tpu_sc as plsc

_NUM_CORES = 2
_NUM_SUBCORES = 16
_NUM_WORKERS = _NUM_CORES * _NUM_SUBCORES


@functools.lru_cache(maxsize=None)
def _make_gather(B, D, chunk):
    rows_per_worker = B // _NUM_WORKERS
    nchunks = rows_per_worker // chunk
    assert nchunks * chunk == rows_per_worker

    mesh = plsc.VectorSubcoreMesh(core_axis_name="c", subcore_axis_name="s")

    @functools.partial(
        pl.kernel,
        mesh=mesh,
        out_type=jax.ShapeDtypeStruct((B, D), jnp.float32),
        scratch_types=[
            pltpu.VMEM((chunk,), jnp.int32),
            pltpu.VMEM((chunk, D), jnp.float32),
            pltpu.SemaphoreType.DMA,
        ],
        compiler_params=pltpu.CompilerParams(use_tc_tiling_on_sc=False),
    )
    def body(idx_hbm, table_hbm, out_hbm, idx_v, rows_v, sem):
        wid = lax.axis_index("s") * _NUM_CORES + lax.axis_index("c")
        base_w = wid * rows_per_worker
        for c in range(nchunks):
            base = base_w + c * chunk
            pltpu.sync_copy(idx_hbm.at[pl.ds(base, chunk)], idx_v)
            pltpu.async_copy(table_hbm.at[idx_v], rows_v, sem).wait()
            pltpu.sync_copy(rows_v, out_hbm.at[pl.ds(base, chunk)])

    return body


def kernel(indices, table):
    Bq, K = indices.shape
    V, D = table.shape
    B = Bq * K
    flat = indices.reshape(B).astype(jnp.int32)
    out = _make_gather(B, D, 1664)(flat, table)
    return out.reshape(Bq, K, D)


# speedup vs baseline: 1.5759x; 1.0090x over previous
import functools

import jax
import jax.numpy as jnp
from jax import lax
from jax.experimental import pallas as pl
from jax.experimental.pallas import tpu as pltpu
from jax.experimental.pallas import tpu_sc as plsc

_NUM_CORES = 2
_NUM_SUBCORES = 16
_NUM_WORKERS = _NUM_CORES * _NUM_SUBCORES


@functools.lru_cache(maxsize=None)
def _make_gather(B, D, chunk, nbuf):
    rows_per_worker = B // _NUM_WORKERS
    nchunks = rows_per_worker // chunk
    assert nchunks * chunk == rows_per_worker
    assert chunk % 8 == 0

    mesh = plsc.VectorSubcoreMesh(core_axis_name="c", subcore_axis_name="s")

    @functools.partial(
        pl.kernel,
        mesh=mesh,
        out_type=jax.ShapeDtypeStruct((B, D), jnp.float32),
        scratch_types=[
            pltpu.VMEM((rows_per_worker,), jnp.int32),
            *[pltpu.VMEM((chunk, D), jnp.float32) for _ in range(nbuf)],
            *[pltpu.SemaphoreType.DMA for _ in range(2 * nbuf)],
        ],
        compiler_params=pltpu.CompilerParams(use_tc_tiling_on_sc=False),
    )
    def body(idx_hbm, table_hbm, out_hbm, idx_v, *rest):
        bufs = rest[:nbuf]
        gsems = rest[nbuf:2 * nbuf]
        wsems = rest[2 * nbuf:]
        wid = lax.axis_index("s") * _NUM_CORES + lax.axis_index("c")
        base_w = wid * rows_per_worker
        pltpu.sync_copy(idx_hbm.at[pl.ds(base_w, rows_per_worker)], idx_v)

        gh = [None] * nbuf
        wh = [None] * nbuf

        def drain(d):
            bd = d % nbuf
            gh[bd].wait()
            wh[bd] = pltpu.async_copy(
                bufs[bd], out_hbm.at[pl.ds(base_w + d * chunk, chunk)],
                wsems[bd])

        for c in range(nchunks):
            b = c % nbuf
            if wh[b] is not None:
                wh[b].wait()
            gh[b] = pltpu.async_copy(
                table_hbm.at[idx_v.at[pl.ds(c * chunk, chunk)]], bufs[b],
                gsems[b])
            d = c - (nbuf - 1)
            if d >= 0:
                drain(d)
        for d in range(max(0, nchunks - (nbuf - 1)), nchunks):
            drain(d)
        for h in wh:
            if h is not None:
                h.wait()

    return body


def kernel(indices, table):
    Bq, K = indices.shape
    V, D = table.shape
    B = Bq * K
    flat = indices.reshape(B).astype(jnp.int32)
    out = _make_gather(B, D, 832, 4)(flat, table)
    return out.reshape(Bq, K, D)
